# Initial kernel scaffold; baseline (speedup 1.0000x reference)
import functools

import jax
import jax.numpy as jnp
from jax import lax
from jax.experimental import pallas as pl
from jax.experimental.pallas import tpu as pltpu
from jax.experimental.pallas import tpu_sc as plsc

N, E, D = 10000, 320000, 128
NC, NS, L = 2, 16, 16
NW = NC * NS
CH = 128
EPW = ((E // NW + CH - 1) // CH) * CH
NCHK = EPW // CH
NPAD = 10240
RPS = NPAD // NS


def _make_sc_agg(with_deg: bool):
    mesh = plsc.VectorSubcoreMesh(core_axis_name="c", subcore_axis_name="s")
    out_type = [jax.ShapeDtypeStruct((NC, NPAD, D), jnp.float32)]
    scratch = [
        pltpu.VMEM((NCHK, CH), jnp.int32),
        pltpu.VMEM((NCHK, CH), jnp.int32),
        pltpu.VMEM((CH, D), jnp.float32),
        pltpu.VMEM((CH, D), jnp.float32),
        pltpu.VMEM_SHARED((NPAD, D), jnp.float32),
        pltpu.SemaphoreType.DMA,
        pltpu.SemaphoreType.DMA,
    ]
    if with_deg:
        out_type.append(jax.ShapeDtypeStruct((NC, NPAD, 8), jnp.float32))
        scratch += [
            pltpu.VMEM((CH, 8), jnp.float32),
            pltpu.VMEM_SHARED((NPAD, 8), jnp.float32),
        ]

    def body(h_hbm, srcw, dstw, zrow, zdeg, ones_h, *rest):
        if with_deg:
            (out_p, deg_p, src_v, dst_v, rows_a, rows_b, acc_sh, sem_a, sem_b,
             ones_v, deg_sh) = rest
        else:
            (out_p, src_v, dst_v, rows_a, rows_b, acc_sh, sem_a, sem_b) = rest
        c = lax.axis_index("c")
        s = lax.axis_index("s")
        wid = c * NS + s
        r0 = s * RPS
        pltpu.sync_copy(srcw.at[wid], src_v)
        pltpu.sync_copy(dstw.at[wid], dst_v)
        pltpu.sync_copy(zrow.at[pl.ds(r0, RPS)], acc_sh.at[pl.ds(r0, RPS)])
        if with_deg:
            pltpu.sync_copy(ones_h, ones_v)
            pltpu.sync_copy(zdeg.at[pl.ds(r0, RPS)], deg_sh.at[pl.ds(r0, RPS)])
        plsc.subcore_barrier()

        pltpu.async_copy(h_hbm.at[src_v.at[0]], rows_a, sem_a)

        def chunk(j, _):
            even = lax.rem(j, 2) == 0

            def do(rows_cur, sem_cur, rows_nxt, sem_nxt):
                pltpu.async_copy(h_hbm.at[src_v.at[j + 1]], rows_nxt, sem_nxt)
                pltpu.make_async_copy(h_hbm.at[src_v.at[j]], rows_cur,
                                      sem_cur).wait()
                pltpu.sync_copy(rows_cur, acc_sh.at[dst_v.at[j]], add=True)
                if with_deg:
                    pltpu.sync_copy(ones_v, deg_sh.at[dst_v.at[j]], add=True)

            lax.cond(even,
                     lambda: do(rows_a, sem_a, rows_b, sem_b),
                     lambda: do(rows_b, sem_b, rows_a, sem_a))
            return 0

        lax.fori_loop(0, NCHK - 1, chunk, 0, unroll=False)
        j = NCHK - 1
        even = (j % 2) == 0

        def last(rows_cur, sem_cur):
            pltpu.make_async_copy(h_hbm.at[src_v.at[j]], rows_cur,
                                  sem_cur).wait()
            pltpu.sync_copy(rows_cur, acc_sh.at[dst_v.at[j]], add=True)
            if with_deg:
                pltpu.sync_copy(ones_v, deg_sh.at[dst_v.at[j]], add=True)

        if even:
            last(rows_a, sem_a)
        else:
            last(rows_b, sem_b)
        plsc.subcore_barrier()
        pltpu.sync_copy(acc_sh.at[pl.ds(r0, RPS)], out_p.at[c, pl.ds(r0, RPS)])
        if with_deg:
            pltpu.sync_copy(deg_sh.at[pl.ds(r0, RPS)],
                            deg_p.at[c, pl.ds(r0, RPS)])

    return pl.kernel(body, out_type=tuple(out_type), mesh=mesh,
                     scratch_types=scratch,
                     name="sc_segsum_deg" if with_deg else "sc_segsum")


_sc_agg_deg = _make_sc_agg(True)
_sc_agg = _make_sc_agg(False)


RB = 1000

_SQRT_HALF = 0.7071067811865476


def _gelu(x):
    return 0.5 * x * (1.0 + lax.erf(x * _SQRT_HALF))


def _tc_in_body(x_ref, w_ref, b_ref, o_ref):
    t = jnp.dot(x_ref[...], w_ref[...], preferred_element_type=jnp.float32)
    o_ref[...] = _gelu(t + b_ref[...])


def _tc_in(x, w, b):
    return pl.pallas_call(
        _tc_in_body,
        grid=(N // RB,),
        in_specs=[
            pl.BlockSpec((RB, D), lambda i: (i, 0)),
            pl.BlockSpec((D, D), lambda i: (0, 0)),
            pl.BlockSpec((1, D), lambda i: (0, 0)),
        ],
        out_specs=pl.BlockSpec((RB, D), lambda i: (i, 0)),
        out_shape=jax.ShapeDtypeStruct((N, D), jnp.float32),
    )(x, w, b)


def _tc_layer_body(final, p_ref, dg_ref, h_ref, wl_ref, bl_ref, wr_ref,
                   g_ref, be_ref, *rest):
    if final:
        wo_ref, bo_ref, o_ref = rest
    else:
        (o_ref,) = rest
    agg = p_ref[0] + p_ref[1]
    deg = dg_ref[0, :, 0:1] + dg_ref[1, :, 0:1]
    mean = agg * (1.0 / jnp.maximum(deg, 1.0))
    t = (jnp.dot(mean, wl_ref[...], preferred_element_type=jnp.float32)
         + jnp.dot(h_ref[...], wr_ref[...], preferred_element_type=jnp.float32)
         + bl_ref[...])
    mu = jnp.mean(t, axis=-1, keepdims=True)
    var = jnp.mean((t - mu) ** 2, axis=-1, keepdims=True)
    y = (t - mu) * lax.rsqrt(var + 1e-5) * g_ref[...] + be_ref[...]
    h_new = _gelu(y) + h_ref[...]
    if final:
        o_ref[...] = (jnp.dot(h_new, wo_ref[...],
                              preferred_element_type=jnp.float32)
                      + bo_ref[...])
    else:
        o_ref[...] = h_new


def _tc_layer(parts, degs, h, wl, bl, wr, g, be, wo=None, bo=None):
    final = wo is not None
    in_specs = [
        pl.BlockSpec((NC, RB, D), lambda i: (0, i, 0)),
        pl.BlockSpec((NC, RB, 8), lambda i: (0, i, 0)),
        pl.BlockSpec((RB, D), lambda i: (i, 0)),
        pl.BlockSpec((D, D), lambda i: (0, 0)),
        pl.BlockSpec((1, D), lambda i: (0, 0)),
        pl.BlockSpec((D, D), lambda i: (0, 0)),
        pl.BlockSpec((1, D), lambda i: (0, 0)),
        pl.BlockSpec((1, D), lambda i: (0, 0)),
    ]
    args = [parts, degs, h, wl, bl, wr, g, be]
    if final:
        in_specs += [pl.BlockSpec((D, D), lambda i: (0, 0)),
                     pl.BlockSpec((1, D), lambda i: (0, 0))]
        args += [wo, bo]
    return pl.pallas_call(
        functools.partial(_tc_layer_body, final),
        grid=(N // RB,),
        in_specs=in_specs,
        out_specs=pl.BlockSpec((RB, D), lambda i: (i, 0)),
        out_shape=jax.ShapeDtypeStruct((N, D), jnp.float32),
    )(*args)


def kernel(x, edge_index, W_in, b_in, Wl0, bl0, Wr0, g0, be0, Wl1, bl1, Wr1,
           g1, be1, Wl2, bl2, Wr2, g2, be2, W_out, b_out):
    src, dst = edge_index[0], edge_index[1]
    pad = NW * EPW - E
    srcw = jnp.concatenate(
        [src, jnp.zeros((pad,), jnp.int32)]).reshape(NW, NCHK, CH)
    dstw = jnp.concatenate(
        [dst, jnp.full((pad,), N, jnp.int32)]).reshape(NW, NCHK, CH)
    zrow = jnp.zeros((NPAD, D), jnp.float32)
    zdeg = jnp.zeros((NPAD, 8), jnp.float32)
    ones_h = jnp.ones((CH, 8), jnp.float32)

    h = _tc_in(x, W_in, b_in.reshape(1, D))
    parts, degs = _sc_agg_deg(h, srcw, dstw, zrow, zdeg, ones_h)
    h = _tc_layer(parts, degs, h, Wl0, bl0.reshape(1, D), Wr0,
                  g0.reshape(1, D), be0.reshape(1, D))
    (parts,) = _sc_agg(h, srcw, dstw, zrow, zdeg, ones_h)
    h = _tc_layer(parts, degs, h, Wl1, bl1.reshape(1, D), Wr1,
                  g1.reshape(1, D), be1.reshape(1, D))
    (parts,) = _sc_agg(h, srcw, dstw, zrow, zdeg, ones_h)
    out = _tc_layer(parts, degs, h, Wl2, bl2.reshape(1, D), Wr2,
                    g2.reshape(1, D), be2.reshape(1, D),
                    W_out, b_out.reshape(1, D))
    return out


# speedup vs baseline: 5.6118x; 5.6118x over previous
import functools

import jax
import jax.numpy as jnp
from jax import lax
from jax.experimental import pallas as pl
from jax.experimental.pallas import tpu as pltpu
from jax.experimental.pallas import tpu_sc as plsc

N, E, D = 10000, 320000, 128
NC, NS = 2, 16
NW = NC * NS
NH = N // NC
CH = 128
VR = 16
EPW_A = ((E // NS + CH - 1) // CH) * CH
NCHK_A = EPW_A // CH
EPW_D = ((E // NW + CH - 1) // CH) * CH
NCHK_D = EPW_D // CH
NPH = 5248
RPH = NPH // NS
NDUM = 240
NPAD = 10240
RPS = NPAD // NS


def _make_sc_agg():
    mesh = plsc.VectorSubcoreMesh(core_axis_name="c", subcore_axis_name="s")
    scratch = [
        pltpu.VMEM((NCHK_A, CH), jnp.int32),
        pltpu.VMEM((NCHK_A, CH), jnp.int32),
        pltpu.VMEM((CH,), jnp.int32),
        pltpu.VMEM((CH, D), jnp.float32),
        pltpu.VMEM((CH, D), jnp.float32),
        pltpu.VMEM_SHARED((NPH, D), jnp.float32),
        pltpu.SemaphoreType.DMA,
        pltpu.SemaphoreType.DMA,
    ]

    def body(h_hbm, srcw, dstw, zrow, out_p, src_v, dst_v, dloc_v, rows_a,
             rows_b, acc_sh, sem_a, sem_b):
        c = lax.axis_index("c")
        s = lax.axis_index("s")
        r0 = s * RPH
        lo = c * NH
        pltpu.sync_copy(srcw.at[s], src_v)
        pltpu.sync_copy(dstw.at[s], dst_v)
        pltpu.sync_copy(zrow.at[pl.ds(r0, RPH)], acc_sh.at[pl.ds(r0, RPH)])
        plsc.subcore_barrier()

        def remap(j):
            for k in range(CH // VR):
                d = dst_v[j, pl.ds(k * VR, VR)]
                dl = d - lo
                mine = (dl >= 0) & (dl < NH)
                dummy = NH + (j * (CH // VR) + k) % NDUM
                dloc_v[pl.ds(k * VR, VR)] = jnp.where(mine, dl, dummy)

        pltpu.async_copy(h_hbm.at[src_v.at[0]], rows_a, sem_a)

        def chunk(j, _):
            even = lax.rem(j, 2) == 0

            def do(rows_cur, sem_cur, rows_nxt, sem_nxt):
                pltpu.async_copy(h_hbm.at[src_v.at[j + 1]], rows_nxt, sem_nxt)
                remap(j)
                pltpu.make_async_copy(h_hbm.at[src_v.at[j]], rows_cur,
                                      sem_cur).wait()
                pltpu.sync_copy(rows_cur, acc_sh.at[dloc_v], add=True)

            lax.cond(even,
                     lambda: do(rows_a, sem_a, rows_b, sem_b),
                     lambda: do(rows_b, sem_b, rows_a, sem_a))
            return 0

        lax.fori_loop(0, NCHK_A - 1, chunk, 0, unroll=False)
        j = NCHK_A - 1
        rows_cur, sem_cur = (rows_a, sem_a) if j % 2 == 0 else (rows_b, sem_b)
        remap(j)
        pltpu.make_async_copy(h_hbm.at[src_v.at[j]], rows_cur, sem_cur).wait()
        pltpu.sync_copy(rows_cur, acc_sh.at[dloc_v], add=True)
        plsc.subcore_barrier()
        pltpu.sync_copy(acc_sh.at[pl.ds(r0, RPH)], out_p.at[c, pl.ds(r0, RPH)])

    return pl.kernel(body,
                     out_type=jax.ShapeDtypeStruct((NC, NPH, D), jnp.float32),
                     mesh=mesh, scratch_types=scratch, name="sc_segsum")


def _make_sc_deg():
    mesh = plsc.VectorSubcoreMesh(core_axis_name="c", subcore_axis_name="s")
    scratch = [
        pltpu.VMEM((NCHK_D, CH), jnp.int32),
        pltpu.VMEM((NPAD,), jnp.float32),
    ]

    def body(dstw, zdeg, deg_p, dst_v, deg_v):
        c = lax.axis_index("c")
        s = lax.axis_index("s")
        wid = c * NS + s
        pltpu.sync_copy(dstw.at[wid], dst_v)
        pltpu.sync_copy(zdeg, deg_v)
        ones = jnp.ones((VR,), jnp.float32)

        def chunk(j, _):
            for k in range(CH // VR):
                idx = dst_v[j, pl.ds(k * VR, VR)]
                plsc.addupdate_scatter(deg_v, [idx], ones)
            return 0

        lax.fori_loop(0, NCHK_D, chunk, 0, unroll=False)
        pltpu.sync_copy(deg_v, deg_p.at[wid])

    return pl.kernel(body,
                     out_type=jax.ShapeDtypeStruct((NW, NPAD), jnp.float32),
                     mesh=mesh, scratch_types=scratch, name="sc_deg",
                     compiler_params=pltpu.CompilerParams(
                         needs_layout_passes=False))


_sc_agg = _make_sc_agg()
_sc_deg = _make_sc_deg()


RB = 1000
NBH = NH // RB

_SQRT_HALF = 0.7071067811865476


def _gelu(x):
    return 0.5 * x * (1.0 + lax.erf(x * _SQRT_HALF))


RB2 = 1024


def _tc_deginv_body(dg_ref, o_ref):
    dsum = jnp.sum(dg_ref[...], axis=0)
    inv = 1.0 / jnp.maximum(dsum, 1.0)
    o_ref[...] = jnp.broadcast_to(inv, (8, RB2)).T


def _tc_deginv(degs):
    return pl.pallas_call(
        _tc_deginv_body,
        grid=(NPAD // RB2,),
        in_specs=[pl.BlockSpec((NW, RB2), lambda i: (0, i))],
        out_specs=pl.BlockSpec((RB2, 8), lambda i: (i, 0)),
        out_shape=jax.ShapeDtypeStruct((NPAD, 8), jnp.float32),
    )(degs)


def _tc_in_body(x_ref, w_ref, b_ref, o_ref):
    t = jnp.dot(x_ref[...], w_ref[...], preferred_element_type=jnp.float32)
    o_ref[...] = _gelu(t + b_ref[...])


def _tc_in(x, w, b):
    return pl.pallas_call(
        _tc_in_body,
        grid=(N // RB,),
        in_specs=[
            pl.BlockSpec((RB, D), lambda i: (i, 0)),
            pl.BlockSpec((D, D), lambda i: (0, 0)),
            pl.BlockSpec((1, D), lambda i: (0, 0)),
        ],
        out_specs=pl.BlockSpec((RB, D), lambda i: (i, 0)),
        out_shape=jax.ShapeDtypeStruct((N, D), jnp.float32),
    )(x, w, b)


def _tc_layer_body(final, p_ref, dg_ref, h_ref, wl_ref, bl_ref, wr_ref,
                   g_ref, be_ref, *rest):
    if final:
        wo_ref, bo_ref, o_ref = rest
    else:
        (o_ref,) = rest
    agg = p_ref[0]
    mean = agg * dg_ref[:, 0:1]
    t = (jnp.dot(mean, wl_ref[...], preferred_element_type=jnp.float32)
         + jnp.dot(h_ref[...], wr_ref[...], preferred_element_type=jnp.float32)
         + bl_ref[...])
    mu = jnp.mean(t, axis=-1, keepdims=True)
    var = jnp.mean((t - mu) ** 2, axis=-1, keepdims=True)
    y = (t - mu) * lax.rsqrt(var + 1e-5) * g_ref[...] + be_ref[...]
    h_new = _gelu(y) + h_ref[...]
    if final:
        o_ref[...] = (jnp.dot(h_new, wo_ref[...],
                              preferred_element_type=jnp.float32)
                      + bo_ref[...])
    else:
        o_ref[...] = h_new


def _tc_layer(parts, degs, h, wl, bl, wr, g, be, wo=None, bo=None):
    final = wo is not None
    in_specs = [
        pl.BlockSpec((1, RB, D), lambda i: (i // NBH, i % NBH, 0)),
        pl.BlockSpec((RB, 8), lambda i: (i, 0)),
        pl.BlockSpec((RB, D), lambda i: (i, 0)),
        pl.BlockSpec((D, D), lambda i: (0, 0)),
        pl.BlockSpec((1, D), lambda i: (0, 0)),
        pl.BlockSpec((D, D), lambda i: (0, 0)),
        pl.BlockSpec((1, D), lambda i: (0, 0)),
        pl.BlockSpec((1, D), lambda i: (0, 0)),
    ]
    args = [parts, degs, h, wl, bl, wr, g, be]
    if final:
        in_specs += [pl.BlockSpec((D, D), lambda i: (0, 0)),
                     pl.BlockSpec((1, D), lambda i: (0, 0))]
        args += [wo, bo]
    return pl.pallas_call(
        functools.partial(_tc_layer_body, final),
        grid=(N // RB,),
        in_specs=in_specs,
        out_specs=pl.BlockSpec((RB, D), lambda i: (i, 0)),
        out_shape=jax.ShapeDtypeStruct((N, D), jnp.float32),
    )(*args)


def kernel(x, edge_index, W_in, b_in, Wl0, bl0, Wr0, g0, be0, Wl1, bl1, Wr1,
           g1, be1, Wl2, bl2, Wr2, g2, be2, W_out, b_out):
    src, dst = edge_index[0], edge_index[1]
    pad_a = NS * EPW_A - E
    srcw = jnp.concatenate(
        [src, jnp.zeros((pad_a,), jnp.int32)]).reshape(NS, NCHK_A, CH)
    dstw = jnp.concatenate(
        [dst, jnp.full((pad_a,), -1, jnp.int32)]).reshape(NS, NCHK_A, CH)
    pad_d = NW * EPW_D - E
    dstw_d = jnp.concatenate(
        [dst, jnp.full((pad_d,), N, jnp.int32)]).reshape(NW, NCHK_D, CH)
    zrow = jnp.zeros((NPH, D), jnp.float32)
    zdeg = jnp.zeros((NPAD,), jnp.float32)

    h = _tc_in(x, W_in, b_in.reshape(1, D))
    degs = _tc_deginv(_sc_deg(dstw_d, zdeg))
    parts = _sc_agg(h, srcw, dstw, zrow)
    h = _tc_layer(parts, degs, h, Wl0, bl0.reshape(1, D), Wr0,
                  g0.reshape(1, D), be0.reshape(1, D))
    parts = _sc_agg(h, srcw, dstw, zrow)
    h = _tc_layer(parts, degs, h, Wl1, bl1.reshape(1, D), Wr1,
                  g1.reshape(1, D), be1.reshape(1, D))
    parts = _sc_agg(h, srcw, dstw, zrow)
    out = _tc_layer(parts, degs, h, Wl2, bl2.reshape(1, D), Wr2,
                    g2.reshape(1, D), be2.reshape(1, D),
                    W_out, b_out.reshape(1, D))
    return out
